# baseline (speedup 1.0000x reference)
import functools

import jax
import jax.numpy as jnp
from jax.experimental import pallas as pl
from jax.experimental.pallas import tpu as pltpu

_BB = 512
_K = 8
_NEG = -3.0e38


def _epilogue(acc_ref, b_ref, n_ref, wout_ref, lout_ref, vout_ref, iout_ref,
              E):
    acc = (acc_ref[...] + b_ref[...]).T
    clean = acc[:E, :]
    raw = acc[E:, :]
    std = jnp.maximum(raw, 0.0) + jnp.log1p(jnp.exp(-jnp.abs(raw)))
    logits = clean + n_ref[...].T * std
    lout_ref[...] = logits.T
    rows = jax.lax.broadcasted_iota(jnp.int32, logits.shape, 0)
    work = logits
    mxs, idxs = [], []
    for _ in range(_K):
        mx = jnp.max(work, axis=0, keepdims=True)
        am = jnp.argmax(work, axis=0).astype(jnp.int32)[None, :]
        mxs.append(mx)
        idxs.append(am)
        work = jnp.where(rows == am, _NEG, work)
    m = mxs[0]
    e = jnp.exp(logits - m)
    s = jnp.sum(e, axis=0, keepdims=True)
    inv_s = 1.0 / s
    wout_ref[...] = (e * inv_s).T
    tl = jnp.concatenate(mxs, axis=0)
    vout_ref[...] = (jnp.exp(tl - m) * inv_s).T
    iout_ref[...] = jnp.concatenate(idxs, axis=0).T


def _body(x_ref, w_ref, b_ref, n_ref, wout_ref, lout_ref, vout_ref, iout_ref,
          acc0_ref, acc1_ref, *, E, nb):
    i = pl.program_id(0)

    @pl.when(i < nb)
    def _matmul():
        mm = jnp.dot(x_ref[...], w_ref[...],
                     preferred_element_type=jnp.float32)

        @pl.when(i % 2 == 0)
        def _w0():
            acc0_ref[...] = mm

        @pl.when(i % 2 == 1)
        def _w1():
            acc1_ref[...] = mm

    @pl.when(i > 0)
    def _epi():
        @pl.when(i % 2 == 1)
        def _e0():
            _epilogue(acc0_ref, b_ref, n_ref, wout_ref, lout_ref, vout_ref,
                      iout_ref, E)

        @pl.when(i % 2 == 0)
        def _e1():
            _epilogue(acc1_ref, b_ref, n_ref, wout_ref, lout_ref, vout_ref,
                      iout_ref, E)


def kernel(x, Wg, bg, Wn, bn, noise, top_k):
    B, D = x.shape
    E = Wg.shape[0]
    W = jnp.concatenate([Wg, Wn], axis=0).T
    b2 = jnp.concatenate([bg, bn])[None, :]
    nb = B // _BB
    grid = (nb + 1,)

    def x_map(i):
        return (jnp.minimum(i, nb - 1), 0)

    def prev_map(i):
        return (jnp.maximum(i - 1, 0), 0)

    out = pl.pallas_call(
        functools.partial(_body, E=E, nb=nb),
        grid=grid,
        in_specs=[
            pl.BlockSpec((_BB, D), x_map),
            pl.BlockSpec((D, 2 * E), lambda i: (0, 0)),
            pl.BlockSpec((1, 2 * E), lambda i: (0, 0)),
            pl.BlockSpec((_BB, E), prev_map),
        ],
        out_specs=[
            pl.BlockSpec((_BB, E), prev_map),
            pl.BlockSpec((_BB, E), prev_map),
            pl.BlockSpec((_BB, _K), prev_map),
            pl.BlockSpec((_BB, _K), prev_map),
        ],
        out_shape=[
            jax.ShapeDtypeStruct((B, E), jnp.float32),
            jax.ShapeDtypeStruct((B, E), jnp.float32),
            jax.ShapeDtypeStruct((B, _K), jnp.float32),
            jax.ShapeDtypeStruct((B, _K), jnp.int32),
        ],
        scratch_shapes=[pltpu.VMEM((_BB, 2 * E), jnp.float32),
                        pltpu.VMEM((_BB, 2 * E), jnp.float32)],
        compiler_params=pltpu.CompilerParams(
            dimension_semantics=("arbitrary",)),
    )(x, W, b2, noise)
    weights, logits, top_k_vals, top_k_indices = out
    return (weights, logits, top_k_vals, top_k_indices)


# speedup vs baseline: 1.0100x; 1.0100x over previous
import functools

import jax
import jax.numpy as jnp
from jax.experimental import pallas as pl
from jax.experimental.pallas import tpu as pltpu

_BB = 1024
_K = 8
_NEG = -3.0e38


def _epilogue(acc_ref, b_ref, n_ref, wout_ref, lout_ref, vout_ref, iout_ref,
              E):
    acc = (acc_ref[...] + b_ref[...]).T
    clean = acc[:E, :]
    raw = acc[E:, :]
    std = jnp.maximum(raw, 0.0) + jnp.log1p(jnp.exp(-jnp.abs(raw)))
    logits = clean + n_ref[...].T * std
    lout_ref[...] = logits.T
    rows = jax.lax.broadcasted_iota(jnp.int32, logits.shape, 0)
    work = logits
    mxs, idxs = [], []
    for _ in range(_K):
        mx = jnp.max(work, axis=0, keepdims=True)
        am = jnp.argmax(work, axis=0).astype(jnp.int32)[None, :]
        mxs.append(mx)
        idxs.append(am)
        work = jnp.where(rows == am, _NEG, work)
    m = mxs[0]
    e = jnp.exp(logits - m)
    s = jnp.sum(e, axis=0, keepdims=True)
    inv_s = 1.0 / s
    wout_ref[...] = (e * inv_s).T
    tl = jnp.concatenate(mxs, axis=0)
    vout_ref[...] = (jnp.exp(tl - m) * inv_s).T
    iout_ref[...] = jnp.concatenate(idxs, axis=0).T


def _body(x_ref, w_ref, b_ref, n_ref, wout_ref, lout_ref, vout_ref, iout_ref,
          acc0_ref, acc1_ref, *, E, nb):
    i = pl.program_id(0)

    @pl.when(i < nb)
    def _matmul():
        mm = jnp.dot(x_ref[...], w_ref[...],
                     preferred_element_type=jnp.float32)

        @pl.when(i % 2 == 0)
        def _w0():
            acc0_ref[...] = mm

        @pl.when(i % 2 == 1)
        def _w1():
            acc1_ref[...] = mm

    @pl.when(i > 0)
    def _epi():
        @pl.when(i % 2 == 1)
        def _e0():
            _epilogue(acc0_ref, b_ref, n_ref, wout_ref, lout_ref, vout_ref,
                      iout_ref, E)

        @pl.when(i % 2 == 0)
        def _e1():
            _epilogue(acc1_ref, b_ref, n_ref, wout_ref, lout_ref, vout_ref,
                      iout_ref, E)


def kernel(x, Wg, bg, Wn, bn, noise, top_k):
    B, D = x.shape
    E = Wg.shape[0]
    W = jnp.concatenate([Wg, Wn], axis=0).T
    b2 = jnp.concatenate([bg, bn])[None, :]
    nb = B // _BB
    grid = (nb + 1,)

    def x_map(i):
        return (jnp.minimum(i, nb - 1), 0)

    def prev_map(i):
        return (jnp.maximum(i - 1, 0), 0)

    out = pl.pallas_call(
        functools.partial(_body, E=E, nb=nb),
        grid=grid,
        in_specs=[
            pl.BlockSpec((_BB, D), x_map),
            pl.BlockSpec((D, 2 * E), lambda i: (0, 0)),
            pl.BlockSpec((1, 2 * E), lambda i: (0, 0)),
            pl.BlockSpec((_BB, E), prev_map),
        ],
        out_specs=[
            pl.BlockSpec((_BB, E), prev_map),
            pl.BlockSpec((_BB, E), prev_map),
            pl.BlockSpec((_BB, _K), prev_map),
            pl.BlockSpec((_BB, _K), prev_map),
        ],
        out_shape=[
            jax.ShapeDtypeStruct((B, E), jnp.float32),
            jax.ShapeDtypeStruct((B, E), jnp.float32),
            jax.ShapeDtypeStruct((B, _K), jnp.float32),
            jax.ShapeDtypeStruct((B, _K), jnp.int32),
        ],
        scratch_shapes=[pltpu.VMEM((_BB, 2 * E), jnp.float32),
                        pltpu.VMEM((_BB, 2 * E), jnp.float32)],
        compiler_params=pltpu.CompilerParams(
            dimension_semantics=("arbitrary",)),
    )(x, W, b2, noise)
    weights, logits, top_k_vals, top_k_indices = out
    return (weights, logits, top_k_vals, top_k_indices)
